# baseline (speedup 1.0000x reference)
import functools

import jax
import jax.numpy as jnp
from jax import lax
from jax.experimental import pallas as pl
from jax.experimental.pallas import tpu as pltpu
from jax.experimental.pallas import tpu_sc as plsc

VOCAB = 100000
HID = 768
B = 4
S = 8192
EPS = 1e-12

N = B * S
NC = 2
NS = 16
NW = NC * NS
TPW = N // NW
CH = 32
NCH = TPW // CH
L = 16
NV = HID // L
TG = 8
JU = 4


def _tec_body(ids_hbm, tt_hbm, word_hbm, pos_hbm, type_hbm, gamma_hbm,
              beta_hbm, out_hbm, idx_v, ttv, wbuf, pbuf, stat_v,
              type_v, g_v, b_v, sg0, sg1, sp0, sp1, so0, so1, si0, si1):
    cid = lax.axis_index("c")
    sid = lax.axis_index("s")
    wid = sid * NC + cid
    base = wid * TPW
    sg = (sg0, sg1)
    sp = (sp0, sp1)
    so = (so0, so1)
    si = (si0, si1)

    pltpu.sync_copy(type_hbm, type_v)
    pltpu.sync_copy(gamma_hbm, g_v)
    pltpu.sync_copy(beta_hbm, b_v)

    def tok0_of(c):
        return jnp.minimum(base + c * CH, N - CH)

    def issue_ids(c, b):
        t0 = tok0_of(c)
        pltpu.async_copy(ids_hbm.at[pl.ds(t0, CH)], idx_v.at[b], si[b])
        pltpu.async_copy(tt_hbm.at[pl.ds(t0, CH)], ttv.at[b], si[b])

    def wait_ids(b):
        pltpu.make_async_copy(ids_hbm.at[pl.ds(0, CH)], idx_v.at[b],
                              si[b]).wait()
        pltpu.make_async_copy(tt_hbm.at[pl.ds(0, CH)], ttv.at[b],
                              si[b]).wait()

    def issue_pos(c, b):
        p0 = lax.rem(tok0_of(c), S)
        pltpu.async_copy(pos_hbm.at[pl.ds(p0, CH)], pbuf.at[b], sp[b])

    def wait_pos(b):
        pltpu.make_async_copy(pos_hbm.at[pl.ds(0, CH)], pbuf.at[b],
                              sp[b]).wait()

    def issue_gather(b):
        pltpu.async_copy(word_hbm.at[idx_v.at[b]], wbuf.at[b], sg[b])

    def wait_gather(b):
        pltpu.make_async_copy(word_hbm.at[idx_v.at[b]], wbuf.at[b],
                              sg[b]).wait()

    def issue_out(c, b):
        pltpu.async_copy(pbuf.at[b], out_hbm.at[pl.ds(base + c * CH, CH)],
                         so[b])

    def wait_out(b):
        pltpu.make_async_copy(pbuf.at[b], out_hbm.at[pl.ds(0, CH)],
                              so[b]).wait()

    lane = lax.iota(jnp.int32, L)

    def pass1(b):
        for t0g in range(0, CH, TG):
            tth = ttv[b, pl.ds((t0g // L) * L, L)]
            tt_nz = tth != 0
            toff = t0g % L
            fvs = []
            for t in range(TG):
                m = (lane == (toff + t)) & tt_nz
                fvs.append(
                    plsc.all_reduce_population_count(m).astype(jnp.float32))
            zero = [jnp.zeros((L,), jnp.float32) for _ in range(TG)]

            def jstep(j0, carry):
                acc = list(carry)
                for dj in range(JU):
                    sl = pl.ds((j0 + dj) * L, L)
                    ty0 = type_v[0, sl]
                    tyd = type_v[1, sl] - ty0
                    for t in range(TG):
                        r = t0g + t
                        x = (wbuf[b, r, sl] + pbuf[b, r, sl]
                             + (ty0 + fvs[t] * tyd))
                        acc[t] = acc[t] + x
                        acc[TG + t] = acc[TG + t] + x * x
                        wbuf[b, r, sl] = x
                return tuple(acc)

            acc = pl.loop(0, NV, step=JU, init_carry=tuple(zero + zero))(
                jstep)
            for t in range(TG):
                s1 = jnp.sum(acc[t])
                s2 = jnp.sum(acc[TG + t])
                mean = s1 * (1.0 / HID)
                var = s2 * (1.0 / HID) - mean * mean + EPS
                bits = lax.bitcast_convert_type(var, jnp.int32)
                y = lax.bitcast_convert_type(
                    jnp.int32(0x5F3759DF) - (bits >> 1), jnp.float32)
                for _ in range(3):
                    y = y * (1.5 - 0.5 * var * y * y)
                stat_v[0, t0g + t, :] = jnp.full((L,), mean, jnp.float32)
                stat_v[1, t0g + t, :] = jnp.full((L,), y, jnp.float32)

    def pass2(b):
        for t0g in range(0, CH, TG):
            mg = [stat_v[0, t0g + t, :] for t in range(TG)]
            rg = [stat_v[1, t0g + t, :] for t in range(TG)]

            def jstep(j0):
                for dj in range(JU):
                    sl = pl.ds((j0 + dj) * L, L)
                    gv = g_v[sl]
                    bv = b_v[sl]
                    for t in range(TG):
                        r = t0g + t
                        pbuf[b, r, sl] = ((wbuf[b, r, sl] - mg[t])
                                          * rg[t] * gv + bv)

            pl.loop(0, NV, step=JU)(jstep)

    issue_ids(0, 0)
    issue_ids(1, 1)
    wait_ids(0)
    issue_gather(0)
    issue_pos(0, 0)

    @pl.loop(0, NCH, step=2)
    def pair(c0):
        for bb in range(2):
            c = c0 + bb
            cur, nxt = bb, 1 - bb
            wait_gather(cur)
            wait_pos(cur)
            wait_ids(nxt)
            issue_gather(nxt)
            pass1(cur)
            if bb == 0:
                @pl.when(c0 >= 2)
                def _wait_prev_out():
                    wait_out(nxt)
            else:
                wait_out(nxt)
            issue_pos(c + 1, nxt)
            pass2(cur)
            issue_out(c, cur)
            issue_ids(c + 2, cur)

    wait_gather(0)
    wait_pos(0)
    wait_ids(1)
    wait_out(1)


@jax.jit
def _bert_embed_sc(ids_flat, tt_flat, word_table, pos_table, type_table,
                   ln_gamma, ln_beta):
    mesh = plsc.VectorSubcoreMesh(core_axis_name="c", subcore_axis_name="s")
    kern = functools.partial(
        pl.kernel,
        mesh=mesh,
        compiler_params=pltpu.CompilerParams(needs_layout_passes=False),
        out_type=jax.ShapeDtypeStruct((N, HID), jnp.float32),
        scratch_types=[
            pltpu.VMEM((2, CH), jnp.int32),
            pltpu.VMEM((2, CH), jnp.int32),
            pltpu.VMEM((2, CH, HID), jnp.float32),
            pltpu.VMEM((2, CH, HID), jnp.float32),
            pltpu.VMEM((2, CH, L), jnp.float32),
            pltpu.VMEM((2, HID), jnp.float32),
            pltpu.VMEM((HID,), jnp.float32),
            pltpu.VMEM((HID,), jnp.float32),
            pltpu.SemaphoreType.DMA,
            pltpu.SemaphoreType.DMA,
            pltpu.SemaphoreType.DMA,
            pltpu.SemaphoreType.DMA,
            pltpu.SemaphoreType.DMA,
            pltpu.SemaphoreType.DMA,
            pltpu.SemaphoreType.DMA,
            pltpu.SemaphoreType.DMA,
        ],
    )(_tec_body)
    return kern(ids_flat, tt_flat, word_table, pos_table, type_table,
                ln_gamma, ln_beta)


def kernel(input_ids, token_type_ids, word_table, pos_table, type_table,
           ln_gamma, ln_beta):
    ids_flat = input_ids.reshape(-1).astype(jnp.int32)
    tt_flat = token_type_ids.reshape(-1).astype(jnp.int32)
    out = _bert_embed_sc(ids_flat, tt_flat, word_table, pos_table,
                         type_table, ln_gamma, ln_beta)
    return out.reshape(B, S, HID)


# speedup vs baseline: 1.2860x; 1.2860x over previous
import functools

import jax
import jax.numpy as jnp
from jax import lax
from jax.experimental import pallas as pl
from jax.experimental.pallas import tpu as pltpu
from jax.experimental.pallas import tpu_sc as plsc

VOCAB = 100000
HID = 768
B = 4
S = 8192
EPS = 1e-12

N = B * S
NC = 2
NS = 16
NW = NC * NS
TPW = N // NW
CH = 16
NCH = TPW // CH
L = 16
NV = HID // L
TG = 8


def _tec_body(ids_hbm, tt_hbm, word_hbm, pos_hbm, type_hbm, gamma_hbm,
              beta_hbm, out_hbm, idx_v, ttv, wbuf, pbuf, obuf, tbuf,
              totbuf, mstat, type_v, g_v, b_v, sg0, sg1, sp0, sp1, so0,
              so1, si0, si1):
    cid = lax.axis_index("c")
    sid = lax.axis_index("s")
    wid = sid * NC + cid
    base = wid * TPW
    sg = (sg0, sg1)
    sp = (sp0, sp1)
    so = (so0, so1)
    si = (si0, si1)

    pltpu.sync_copy(type_hbm, type_v)
    pltpu.sync_copy(gamma_hbm, g_v)
    pltpu.sync_copy(beta_hbm, b_v)

    def tok0_of(c):
        return jnp.minimum(base + c * CH, N - CH)

    def issue_ids(c, b):
        t0 = tok0_of(c)
        pltpu.async_copy(ids_hbm.at[pl.ds(t0, CH)], idx_v.at[b], si[b])
        pltpu.async_copy(tt_hbm.at[pl.ds(t0, CH)], ttv.at[b], si[b])

    def wait_ids(b):
        pltpu.make_async_copy(ids_hbm.at[pl.ds(0, CH)], idx_v.at[b],
                              si[b]).wait()
        pltpu.make_async_copy(tt_hbm.at[pl.ds(0, CH)], ttv.at[b],
                              si[b]).wait()

    def issue_pos(c, b):
        p0 = lax.rem(tok0_of(c), S)
        pltpu.async_copy(pos_hbm.at[pl.ds(p0, CH)], pbuf.at[b], sp[b])

    def wait_pos(b):
        pltpu.make_async_copy(pos_hbm.at[pl.ds(0, CH)], pbuf.at[b],
                              sp[b]).wait()

    def issue_gather(b):
        pltpu.async_copy(word_hbm.at[idx_v.at[b]], wbuf.at[b], sg[b])

    def wait_gather(b):
        pltpu.make_async_copy(word_hbm.at[idx_v.at[b]], wbuf.at[b],
                              sg[b]).wait()

    def issue_out(c, b):
        pltpu.async_copy(obuf.at[b], out_hbm.at[pl.ds(base + c * CH, CH)],
                         so[b])

    def wait_out(b):
        pltpu.make_async_copy(obuf.at[b], out_hbm.at[pl.ds(0, CH)],
                              so[b]).wait()

    lane = lax.iota(jnp.int32, L)

    def compute(b):
        ttvec = ttv[b, :]
        tt_nz = ttvec != 0
        for t0g in range(0, CH, TG):
            msks = []
            for t in range(TG):
                m = (lane == (t0g + t)) & tt_nz
                msks.append(plsc.all_reduce_population_count(m) > 0)
            accs = [jnp.zeros((L,), jnp.float32) for _ in range(TG)]
            accq = [jnp.zeros((L,), jnp.float32) for _ in range(TG)]
            for j in range(NV):
                sl = pl.ds(j * L, L)
                ty0 = type_v[0, sl]
                ty1 = type_v[1, sl]
                for t in range(TG):
                    r = t0g + t
                    x = (wbuf[b, r, sl] + pbuf[b, r, sl]
                         + jnp.where(msks[t], ty1, ty0))
                    accs[t] = accs[t] + x
                    accq[t] = accq[t] + x * x
                    obuf[b, r, sl] = x
            for t in range(TG):
                tbuf[pl.ds(t * L, L)] = accs[t]
                tbuf[pl.ds((TG + t) * L, L)] = accq[t]
            base_idx = lane * L
            cols = [plsc.load_gather(tbuf, [base_idx + c]) for c in range(L)]
            while len(cols) > 1:
                cols = [cols[i] + cols[i + 1]
                        for i in range(0, len(cols), 2)]
            tot = cols[0]
            s2v = tot.at[(lane + TG) & (L - 1)].get(
                mode="promise_in_bounds")
            mean8 = tot * (1.0 / HID)
            var8 = jnp.abs(s2v * (1.0 / HID) - mean8 * mean8) + EPS
            bits = plsc.bitcast(var8, jnp.int32)
            y = plsc.bitcast(jnp.int32(0x5F3759DF) - (bits >> 1),
                             jnp.float32)
            for _ in range(3):
                y = y * (1.5 - 0.5 * var8 * y * y)
            means = [mean8.at[jnp.full((L,), t, jnp.int32)].get(
                mode="promise_in_bounds") for t in range(TG)]
            rstds = [y.at[jnp.full((L,), t, jnp.int32)].get(
                mode="promise_in_bounds") for t in range(TG)]
            def p2step(j0):
                for dj in range(8):
                    sl = pl.ds((j0 + dj) * L, L)
                    gv = g_v[sl]
                    bv = b_v[sl]
                    for t in range(TG):
                        r = t0g + t
                        obuf[b, r, sl] = ((obuf[b, r, sl] - means[t])
                                          * rstds[t] * gv + bv)

            pl.loop(0, NV, step=8)(p2step)

    issue_ids(0, 0)
    issue_ids(1, 1)
    wait_ids(0)
    issue_gather(0)
    issue_pos(0, 0)

    @pl.loop(0, NCH, step=2)
    def pair(c0):
        for bb in range(2):
            c = c0 + bb
            cur, nxt = bb, 1 - bb
            wait_gather(cur)
            wait_pos(cur)
            wait_ids(nxt)
            issue_gather(nxt)
            issue_pos(c + 1, nxt)
            @pl.when(c0 >= 2)
            def _wait_prev_out():
                wait_out(cur)
            compute(cur)
            issue_out(c, cur)
            issue_ids(c + 2, cur)

    wait_gather(0)
    wait_pos(0)
    wait_ids(1)
    wait_out(0)
    wait_out(1)


@jax.jit
def _bert_embed_sc(ids_flat, tt_flat, word_table, pos_table, type_table,
                   ln_gamma, ln_beta):
    mesh = plsc.VectorSubcoreMesh(core_axis_name="c", subcore_axis_name="s")
    kern = functools.partial(
        pl.kernel,
        mesh=mesh,
        compiler_params=pltpu.CompilerParams(needs_layout_passes=False),
        out_type=jax.ShapeDtypeStruct((N, HID), jnp.float32),
        scratch_types=[
            pltpu.VMEM((2, CH), jnp.int32),
            pltpu.VMEM((2, CH), jnp.int32),
            pltpu.VMEM((2, CH, HID), jnp.float32),
            pltpu.VMEM((2, CH, HID), jnp.float32),
            pltpu.VMEM((2, CH, HID), jnp.float32),
            pltpu.VMEM((2 * TG * L,), jnp.float32),
            pltpu.VMEM((L,), jnp.float32),
            pltpu.VMEM((2 * L,), jnp.float32),
            pltpu.VMEM((2, HID), jnp.float32),
            pltpu.VMEM((HID,), jnp.float32),
            pltpu.VMEM((HID,), jnp.float32),
            pltpu.SemaphoreType.DMA,
            pltpu.SemaphoreType.DMA,
            pltpu.SemaphoreType.DMA,
            pltpu.SemaphoreType.DMA,
            pltpu.SemaphoreType.DMA,
            pltpu.SemaphoreType.DMA,
            pltpu.SemaphoreType.DMA,
            pltpu.SemaphoreType.DMA,
        ],
    )(_tec_body)
    return kern(ids_flat, tt_flat, word_table, pos_table, type_table,
                ln_gamma, ln_beta)


def kernel(input_ids, token_type_ids, word_table, pos_table, type_table,
           ln_gamma, ln_beta):
    ids_flat = input_ids.reshape(-1).astype(jnp.int32)
    tt_flat = token_type_ids.reshape(-1).astype(jnp.int32)
    out = _bert_embed_sc(ids_flat, tt_flat, word_table, pos_table,
                         type_table, ln_gamma, ln_beta)
    return out.reshape(B, S, HID)


# speedup vs baseline: 1.3939x; 1.0840x over previous
import functools

import jax
import jax.numpy as jnp
from jax import lax
from jax.experimental import pallas as pl
from jax.experimental.pallas import tpu as pltpu
from jax.experimental.pallas import tpu_sc as plsc

VOCAB = 100000
HID = 768
B = 4
S = 8192
EPS = 1e-12

N = B * S
NC = 2
NS = 16
NW = NC * NS
TPW = N // NW
CH = 16
NCH = TPW // CH
L = 16
NV = HID // L
TG = 8
BT = 512


def _tec_body(ids_hbm, tt_hbm, word_hbm, pos_hbm, type_hbm, x_hbm, st_hbm,
              idx_v, ttv, wbuf, pbuf, obuf, sbuf, tbuf, type_v,
              sg0, sg1, sp0, sp1, so0, so1, si0, si1):
    cid = lax.axis_index("c")
    sid = lax.axis_index("s")
    wid = sid * NC + cid
    base = wid * TPW
    sg = (sg0, sg1)
    sp = (sp0, sp1)
    so = (so0, so1)
    si = (si0, si1)

    pltpu.sync_copy(type_hbm, type_v)

    def tok0_of(c):
        return jnp.minimum(base + c * CH, N - CH)

    def issue_ids(c, b):
        t0 = tok0_of(c)
        pltpu.async_copy(ids_hbm.at[pl.ds(t0, CH)], idx_v.at[b], si[b])
        pltpu.async_copy(tt_hbm.at[pl.ds(t0, CH)], ttv.at[b], si[b])

    def wait_ids(b):
        pltpu.make_async_copy(ids_hbm.at[pl.ds(0, CH)], idx_v.at[b],
                              si[b]).wait()
        pltpu.make_async_copy(tt_hbm.at[pl.ds(0, CH)], ttv.at[b],
                              si[b]).wait()

    def issue_pos(c, b):
        p0 = lax.rem(tok0_of(c), S)
        pltpu.async_copy(pos_hbm.at[pl.ds(p0, CH)], pbuf.at[b], sp[b])

    def wait_pos(b):
        pltpu.make_async_copy(pos_hbm.at[pl.ds(0, CH)], pbuf.at[b],
                              sp[b]).wait()

    def issue_gather(b):
        pltpu.async_copy(word_hbm.at[idx_v.at[b]], wbuf.at[b], sg[b])

    def wait_gather(b):
        pltpu.make_async_copy(word_hbm.at[idx_v.at[b]], wbuf.at[b],
                              sg[b]).wait()

    def issue_out(c, b):
        t0 = base + c * CH
        pltpu.async_copy(obuf.at[b], x_hbm.at[pl.ds(t0, CH)], so[b])
        pltpu.async_copy(sbuf.at[b], st_hbm.at[pl.ds(2 * t0, 2 * CH)],
                         so[b])

    def wait_out(b):
        pltpu.make_async_copy(obuf.at[b], x_hbm.at[pl.ds(0, CH)],
                              so[b]).wait()
        pltpu.make_async_copy(sbuf.at[b], st_hbm.at[pl.ds(0, 2 * CH)],
                              so[b]).wait()

    lane = lax.iota(jnp.int32, L)

    def compute(b):
        ttvec = ttv[b, :]
        tt_nz = ttvec != 0
        for t0g in range(0, CH, TG):
            msks = []
            for t in range(TG):
                m = (lane == (t0g + t)) & tt_nz
                msks.append(plsc.all_reduce_population_count(m) > 0)
            accs = [jnp.zeros((L,), jnp.float32) for _ in range(TG)]
            accq = [jnp.zeros((L,), jnp.float32) for _ in range(TG)]
            for j in range(NV):
                sl = pl.ds(j * L, L)
                ty0 = type_v[0, sl]
                ty1 = type_v[1, sl]
                for t in range(TG):
                    r = t0g + t
                    x = (wbuf[b, r, sl] + pbuf[b, r, sl]
                         + jnp.where(msks[t], ty1, ty0))
                    accs[t] = accs[t] + x
                    accq[t] = accq[t] + x * x
                    obuf[b, r, sl] = x
            for t in range(TG):
                tbuf[pl.ds(t * L, L)] = accs[t]
                tbuf[pl.ds((TG + t) * L, L)] = accq[t]
            base_idx = lane * L
            cols = [plsc.load_gather(tbuf, [base_idx + c]) for c in range(L)]
            while len(cols) > 1:
                cols = [cols[i] + cols[i + 1]
                        for i in range(0, len(cols), 2)]
            tot = cols[0]
            plsc.store_scatter(sbuf.at[b], [(lane + t0g) * 2], tot,
                               mask=lane < TG)
            plsc.store_scatter(sbuf.at[b], [(lane - TG + t0g) * 2 + 1], tot,
                               mask=lane >= TG)

    issue_ids(0, 0)
    issue_ids(1, 1)
    wait_ids(0)
    issue_gather(0)
    issue_pos(0, 0)

    @pl.loop(0, NCH, step=2)
    def pair(c0):
        for bb in range(2):
            c = c0 + bb
            cur, nxt = bb, 1 - bb
            wait_gather(cur)
            wait_pos(cur)
            wait_ids(nxt)
            issue_gather(nxt)
            issue_pos(c + 1, nxt)
            @pl.when(c0 >= 2)
            def _wait_prev_out():
                wait_out(cur)
            compute(cur)
            issue_out(c, cur)
            issue_ids(c + 2, cur)

    wait_gather(0)
    wait_pos(0)
    wait_ids(1)
    wait_out(0)
    wait_out(1)


def _tc_body(x_ref, s_ref, g_ref, b_ref, o_ref):
    x = x_ref[...]
    mean = s_ref[:, 0:1] * (1.0 / HID)
    var = s_ref[:, 1:2] * (1.0 / HID) - mean * mean
    rstd = lax.rsqrt(var + EPS)
    o_ref[...] = (x - mean) * rstd * g_ref[...] + b_ref[...]


@jax.jit
def _bert_embed(ids_flat, tt_flat, word_table, pos_table, type_table,
                ln_gamma, ln_beta):
    mesh = plsc.VectorSubcoreMesh(core_axis_name="c", subcore_axis_name="s")
    kern = functools.partial(
        pl.kernel,
        mesh=mesh,
        compiler_params=pltpu.CompilerParams(needs_layout_passes=False),
        out_type=[
            jax.ShapeDtypeStruct((N, HID), jnp.float32),
            jax.ShapeDtypeStruct((2 * N,), jnp.float32),
        ],
        scratch_types=[
            pltpu.VMEM((2, CH), jnp.int32),
            pltpu.VMEM((2, CH), jnp.int32),
            pltpu.VMEM((2, CH, HID), jnp.float32),
            pltpu.VMEM((2, CH, HID), jnp.float32),
            pltpu.VMEM((2, CH, HID), jnp.float32),
            pltpu.VMEM((2, 2 * CH), jnp.float32),
            pltpu.VMEM((2 * TG * L,), jnp.float32),
            pltpu.VMEM((2, HID), jnp.float32),
            pltpu.SemaphoreType.DMA,
            pltpu.SemaphoreType.DMA,
            pltpu.SemaphoreType.DMA,
            pltpu.SemaphoreType.DMA,
            pltpu.SemaphoreType.DMA,
            pltpu.SemaphoreType.DMA,
            pltpu.SemaphoreType.DMA,
            pltpu.SemaphoreType.DMA,
        ],
    )(_tec_body)
    x, stats = kern(ids_flat, tt_flat, word_table, pos_table, type_table)
    out = pl.pallas_call(
        _tc_body,
        grid=(N // BT,),
        in_specs=[
            pl.BlockSpec((BT, HID), lambda i: (i, 0)),
            pl.BlockSpec((BT, 2), lambda i: (i, 0)),
            pl.BlockSpec((1, HID), lambda i: (0, 0)),
            pl.BlockSpec((1, HID), lambda i: (0, 0)),
        ],
        out_specs=pl.BlockSpec((BT, HID), lambda i: (i, 0)),
        out_shape=jax.ShapeDtypeStruct((N, HID), jnp.float32),
    )(x, stats.reshape(N, 2), ln_gamma.reshape(1, HID),
      ln_beta.reshape(1, HID))
    return out


def kernel(input_ids, token_type_ids, word_table, pos_table, type_table,
           ln_gamma, ln_beta):
    ids_flat = input_ids.reshape(-1).astype(jnp.int32)
    tt_flat = token_type_ids.reshape(-1).astype(jnp.int32)
    out = _bert_embed(ids_flat, tt_flat, word_table, pos_table,
                      type_table, ln_gamma, ln_beta)
    return out.reshape(B, S, HID)


# speedup vs baseline: 3.4033x; 2.4415x over previous
import functools

import jax
import jax.numpy as jnp
from jax import lax
from jax.experimental import pallas as pl
from jax.experimental.pallas import tpu as pltpu
from jax.experimental.pallas import tpu_sc as plsc

VOCAB = 100000
HID = 768
B = 4
S = 8192
EPS = 1e-12

N = B * S
NC = 2
NS = 16
NW = NC * NS
TPW = N // NW
CH = 64
NCH = TPW // CH
BT = 512


def _tec_body(ids_hbm, word_hbm, x_hbm, idx_v, wbuf,
              sg0, sg1, so0, so1, si0, si1):
    cid = lax.axis_index("c")
    sid = lax.axis_index("s")
    wid = sid * NC + cid
    base = wid * TPW
    sg = (sg0, sg1)
    so = (so0, so1)
    si = (si0, si1)

    def tok0_of(c):
        return jnp.minimum(base + c * CH, N - CH)

    def issue_ids(c, b):
        pltpu.async_copy(ids_hbm.at[pl.ds(tok0_of(c), CH)], idx_v.at[b],
                         si[b])

    def wait_ids(b):
        pltpu.make_async_copy(ids_hbm.at[pl.ds(0, CH)], idx_v.at[b],
                              si[b]).wait()

    def issue_gather(b):
        pltpu.async_copy(word_hbm.at[idx_v.at[b]], wbuf.at[b], sg[b])

    def wait_gather(b):
        pltpu.make_async_copy(word_hbm.at[idx_v.at[b]], wbuf.at[b],
                              sg[b]).wait()

    def issue_out(c, b):
        pltpu.async_copy(wbuf.at[b], x_hbm.at[pl.ds(base + c * CH, CH)],
                         so[b])

    def wait_out(b):
        pltpu.make_async_copy(wbuf.at[b], x_hbm.at[pl.ds(0, CH)],
                              so[b]).wait()

    issue_ids(0, 0)
    issue_ids(1, 1)
    wait_ids(0)
    issue_gather(0)

    @pl.loop(0, NCH, step=2)
    def pair(c0):
        for bb in range(2):
            c = c0 + bb
            cur, nxt = bb, 1 - bb
            wait_gather(cur)
            wait_ids(nxt)
            if bb == 0:
                @pl.when(c0 >= 2)
                def _wait_prev_out():
                    wait_out(nxt)
            else:
                wait_out(nxt)
            issue_gather(nxt)
            issue_out(c, cur)
            issue_ids(c + 2, cur)

    wait_gather(0)
    wait_ids(1)
    wait_out(1)


def _tc_body(w_ref, p_ref, tt_ref, ty_ref, g_ref, b_ref, o_ref):
    tt = tt_ref[...]
    trow = jnp.where(tt == 0, ty_ref[0:1, :], ty_ref[1:2, :])
    x = w_ref[...] + p_ref[...] + trow
    mean = jnp.mean(x, axis=-1, keepdims=True)
    cx = x - mean
    var = jnp.mean(cx * cx, axis=-1, keepdims=True)
    o_ref[...] = cx * lax.rsqrt(var + EPS) * g_ref[...] + b_ref[...]


@jax.jit
def _bert_embed(ids_flat, tt_flat, word_table, pos_table, type_table,
                ln_gamma, ln_beta):
    mesh = plsc.VectorSubcoreMesh(core_axis_name="c", subcore_axis_name="s")
    kern = functools.partial(
        pl.kernel,
        mesh=mesh,
        compiler_params=pltpu.CompilerParams(needs_layout_passes=False),
        out_type=jax.ShapeDtypeStruct((N, HID), jnp.float32),
        scratch_types=[
            pltpu.VMEM((2, CH), jnp.int32),
            pltpu.VMEM((2, CH, HID), jnp.float32),
            pltpu.SemaphoreType.DMA,
            pltpu.SemaphoreType.DMA,
            pltpu.SemaphoreType.DMA,
            pltpu.SemaphoreType.DMA,
            pltpu.SemaphoreType.DMA,
            pltpu.SemaphoreType.DMA,
        ],
    )(_tec_body)
    w = kern(ids_flat, word_table)
    out = pl.pallas_call(
        _tc_body,
        grid=(N // BT,),
        in_specs=[
            pl.BlockSpec((BT, HID), lambda i: (i, 0)),
            pl.BlockSpec((BT, HID), lambda i: (i % (S // BT), 0)),
            pl.BlockSpec((BT, 1), lambda i: (i, 0)),
            pl.BlockSpec((2, HID), lambda i: (0, 0)),
            pl.BlockSpec((1, HID), lambda i: (0, 0)),
            pl.BlockSpec((1, HID), lambda i: (0, 0)),
        ],
        out_specs=pl.BlockSpec((BT, HID), lambda i: (i, 0)),
        out_shape=jax.ShapeDtypeStruct((N, HID), jnp.float32),
    )(w, pos_table, tt_flat.reshape(N, 1), type_table,
      ln_gamma.reshape(1, HID), ln_beta.reshape(1, HID))
    return out


def kernel(input_ids, token_type_ids, word_table, pos_table, type_table,
           ln_gamma, ln_beta):
    ids_flat = input_ids.reshape(-1).astype(jnp.int32)
    tt_flat = token_type_ids.reshape(-1).astype(jnp.int32)
    out = _bert_embed(ids_flat, tt_flat, word_table, pos_table,
                      type_table, ln_gamma, ln_beta)
    return out.reshape(B, S, HID)


# speedup vs baseline: 3.5426x; 1.0409x over previous
import functools

import jax
import jax.numpy as jnp
from jax import lax
from jax.experimental import pallas as pl
from jax.experimental.pallas import tpu as pltpu
from jax.experimental.pallas import tpu_sc as plsc

VOCAB = 100000
HID = 768
B = 4
S = 8192
EPS = 1e-12

N = B * S
NC = 2
NS = 16
NW = NC * NS
TPW = N // NW
CH = 64
NCH = TPW // CH
BT = 512


def _tec_body(ids_hbm, word_hbm, x_hbm, idx_v, wbuf,
              sg0, sg1, so0, so1, si0, si1):
    cid = lax.axis_index("c")
    sid = lax.axis_index("s")
    wid = sid * NC + cid
    base = wid * TPW
    sg = (sg0, sg1)
    so = (so0, so1)
    si = (si0, si1)

    def tok0_of(c):
        return jnp.minimum(base + c * CH, N - CH)

    def issue_ids(c, b):
        pltpu.async_copy(ids_hbm.at[pl.ds(tok0_of(c), CH)], idx_v.at[b],
                         si[b])

    def wait_ids(b):
        pltpu.make_async_copy(ids_hbm.at[pl.ds(0, CH)], idx_v.at[b],
                              si[b]).wait()

    def issue_gather(b):
        pltpu.async_copy(word_hbm.at[idx_v.at[b]], wbuf.at[b], sg[b])

    def wait_gather(b):
        pltpu.make_async_copy(word_hbm.at[idx_v.at[b]], wbuf.at[b],
                              sg[b]).wait()

    def issue_out(c, b):
        pltpu.async_copy(wbuf.at[b], x_hbm.at[pl.ds(base + c * CH, CH)],
                         so[b])

    def wait_out(b):
        pltpu.make_async_copy(wbuf.at[b], x_hbm.at[pl.ds(0, CH)],
                              so[b]).wait()

    issue_ids(0, 0)
    issue_ids(1, 1)
    wait_ids(0)
    issue_gather(0)

    @pl.loop(0, NCH, step=2)
    def pair(c0):
        for bb in range(2):
            c = c0 + bb
            cur, nxt = bb, 1 - bb
            wait_gather(cur)
            wait_ids(nxt)
            if bb == 0:
                @pl.when(c0 >= 2)
                def _wait_prev_out():
                    wait_out(nxt)
            else:
                wait_out(nxt)
            issue_gather(nxt)
            issue_out(c, cur)
            issue_ids(c + 2, cur)

    wait_gather(0)
    wait_ids(1)
    wait_out(1)


def _tc_body(w_ref, p_ref, tt_ref, ty_ref, g_ref, b_ref, o_ref):
    tt = tt_ref[...]
    trow = jnp.where(tt == 0, ty_ref[0:1, :], ty_ref[1:2, :])
    x = w_ref[...] + p_ref[...] + trow
    mean = jnp.mean(x, axis=-1, keepdims=True)
    cx = x - mean
    var = jnp.mean(cx * cx, axis=-1, keepdims=True)
    o_ref[...] = cx * lax.rsqrt(var + EPS) * g_ref[...] + b_ref[...]


@jax.jit
def _bert_embed(ids_flat, tt_flat, word_table, pos_table, type_table,
                ln_gamma, ln_beta):
    mesh = plsc.VectorSubcoreMesh(core_axis_name="c", subcore_axis_name="s")
    kern = functools.partial(
        pl.kernel,
        mesh=mesh,
        compiler_params=pltpu.CompilerParams(needs_layout_passes=False),
        out_type=jax.ShapeDtypeStruct((N, HID), jnp.float32),
        scratch_types=[
            pltpu.VMEM((2, CH), jnp.int32),
            pltpu.VMEM((2, CH, HID), jnp.float32),
            pltpu.SemaphoreType.DMA,
            pltpu.SemaphoreType.DMA,
            pltpu.SemaphoreType.DMA,
            pltpu.SemaphoreType.DMA,
            pltpu.SemaphoreType.DMA,
            pltpu.SemaphoreType.DMA,
        ],
    )(_tec_body)
    w = kern(ids_flat, word_table)
    out = pl.pallas_call(
        _tc_body,
        grid=(S // BT, B),
        in_specs=[
            pl.BlockSpec((BT, HID), lambda j, b: (b * (S // BT) + j, 0)),
            pl.BlockSpec((BT, HID), lambda j, b: (j, 0)),
            pl.BlockSpec((BT, 1), lambda j, b: (b * (S // BT) + j, 0)),
            pl.BlockSpec((2, HID), lambda j, b: (0, 0)),
            pl.BlockSpec((1, HID), lambda j, b: (0, 0)),
            pl.BlockSpec((1, HID), lambda j, b: (0, 0)),
        ],
        out_specs=pl.BlockSpec((BT, HID), lambda j, b: (b * (S // BT) + j, 0)),
        out_shape=jax.ShapeDtypeStruct((N, HID), jnp.float32),
    )(w, pos_table, tt_flat.reshape(N, 1), type_table,
      ln_gamma.reshape(1, HID), ln_beta.reshape(1, HID))
    return out


def kernel(input_ids, token_type_ids, word_table, pos_table, type_table,
           ln_gamma, ln_beta):
    ids_flat = input_ids.reshape(-1).astype(jnp.int32)
    tt_flat = token_type_ids.reshape(-1).astype(jnp.int32)
    out = _bert_embed(ids_flat, tt_flat, word_table, pos_table,
                      type_table, ln_gamma, ln_beta)
    return out.reshape(B, S, HID)


# speedup vs baseline: 3.7785x; 1.0666x over previous
import functools

import jax
import jax.numpy as jnp
from jax import lax
from jax.experimental import pallas as pl
from jax.experimental.pallas import tpu as pltpu
from jax.experimental.pallas import tpu_sc as plsc

VOCAB = 100000
HID = 768
B = 4
S = 8192
EPS = 1e-12

N = B * S
NC = 2
NS = 16
NW = NC * NS
TPW = N // NW
CH = 64
NCH = TPW // CH
BT = 1024


def _tec_body(ids_hbm, word_hbm, x_hbm, idx_v, wbuf,
              sg0, sg1, so0, so1, si0, si1):
    cid = lax.axis_index("c")
    sid = lax.axis_index("s")
    wid = sid * NC + cid
    base = wid * TPW
    sg = (sg0, sg1)
    so = (so0, so1)
    si = (si0, si1)

    def tok0_of(c):
        return jnp.minimum(base + c * CH, N - CH)

    def issue_ids(c, b):
        pltpu.async_copy(ids_hbm.at[pl.ds(tok0_of(c), CH)], idx_v.at[b],
                         si[b])

    def wait_ids(b):
        pltpu.make_async_copy(ids_hbm.at[pl.ds(0, CH)], idx_v.at[b],
                              si[b]).wait()

    def issue_gather(b):
        pltpu.async_copy(word_hbm.at[idx_v.at[b]], wbuf.at[b], sg[b])

    def wait_gather(b):
        pltpu.make_async_copy(word_hbm.at[idx_v.at[b]], wbuf.at[b],
                              sg[b]).wait()

    def issue_out(c, b):
        pltpu.async_copy(wbuf.at[b], x_hbm.at[pl.ds(base + c * CH, CH)],
                         so[b])

    def wait_out(b):
        pltpu.make_async_copy(wbuf.at[b], x_hbm.at[pl.ds(0, CH)],
                              so[b]).wait()

    issue_ids(0, 0)
    issue_ids(1, 1)
    wait_ids(0)
    issue_gather(0)

    @pl.loop(0, NCH, step=2)
    def pair(c0):
        for bb in range(2):
            c = c0 + bb
            cur, nxt = bb, 1 - bb
            wait_gather(cur)
            wait_ids(nxt)
            if bb == 0:
                @pl.when(c0 >= 2)
                def _wait_prev_out():
                    wait_out(nxt)
            else:
                wait_out(nxt)
            issue_gather(nxt)
            issue_out(c, cur)
            issue_ids(c + 2, cur)

    wait_gather(0)
    wait_ids(1)
    wait_out(1)


def _tc_body(w_ref, p_ref, tt_ref, ty_ref, g_ref, b_ref, o_ref):
    tt = tt_ref[...]
    trow = jnp.where(tt == 0, ty_ref[0:1, :], ty_ref[1:2, :])
    x = w_ref[...] + p_ref[...] + trow
    mean = jnp.mean(x, axis=-1, keepdims=True)
    cx = x - mean
    var = jnp.mean(cx * cx, axis=-1, keepdims=True)
    o_ref[...] = cx * lax.rsqrt(var + EPS) * g_ref[...] + b_ref[...]


@jax.jit
def _bert_embed(ids_flat, tt_flat, word_table, pos_table, type_table,
                ln_gamma, ln_beta):
    mesh = plsc.VectorSubcoreMesh(core_axis_name="c", subcore_axis_name="s")
    kern = functools.partial(
        pl.kernel,
        mesh=mesh,
        compiler_params=pltpu.CompilerParams(needs_layout_passes=False),
        out_type=jax.ShapeDtypeStruct((N, HID), jnp.float32),
        scratch_types=[
            pltpu.VMEM((2, CH), jnp.int32),
            pltpu.VMEM((2, CH, HID), jnp.float32),
            pltpu.SemaphoreType.DMA,
            pltpu.SemaphoreType.DMA,
            pltpu.SemaphoreType.DMA,
            pltpu.SemaphoreType.DMA,
            pltpu.SemaphoreType.DMA,
            pltpu.SemaphoreType.DMA,
        ],
    )(_tec_body)
    w = kern(ids_flat, word_table)
    out = pl.pallas_call(
        _tc_body,
        grid=(S // BT, B),
        in_specs=[
            pl.BlockSpec((BT, HID), lambda j, b: (b * (S // BT) + j, 0)),
            pl.BlockSpec((BT, HID), lambda j, b: (j, 0)),
            pl.BlockSpec((BT, 1), lambda j, b: (b * (S // BT) + j, 0)),
            pl.BlockSpec((2, HID), lambda j, b: (0, 0)),
            pl.BlockSpec((1, HID), lambda j, b: (0, 0)),
            pl.BlockSpec((1, HID), lambda j, b: (0, 0)),
        ],
        out_specs=pl.BlockSpec((BT, HID), lambda j, b: (b * (S // BT) + j, 0)),
        out_shape=jax.ShapeDtypeStruct((N, HID), jnp.float32),
    )(w, pos_table, tt_flat.reshape(N, 1), type_table,
      ln_gamma.reshape(1, HID), ln_beta.reshape(1, HID))
    return out


def kernel(input_ids, token_type_ids, word_table, pos_table, type_table,
           ln_gamma, ln_beta):
    ids_flat = input_ids.reshape(-1).astype(jnp.int32)
    tt_flat = token_type_ids.reshape(-1).astype(jnp.int32)
    out = _bert_embed(ids_flat, tt_flat, word_table, pos_table,
                      type_table, ln_gamma, ln_beta)
    return out.reshape(B, S, HID)


# speedup vs baseline: 3.9461x; 1.0444x over previous
import functools

import jax
import jax.numpy as jnp
from jax import lax
from jax.experimental import pallas as pl
from jax.experimental.pallas import tpu as pltpu
from jax.experimental.pallas import tpu_sc as plsc

VOCAB = 100000
HID = 768
B = 4
S = 8192
EPS = 1e-12

N = B * S
NC = 2
NS = 16
NW = NC * NS
TPW = N // NW
CH = 64
NCH = TPW // CH
BT = 2048


def _tec_body(ids_hbm, word_hbm, x_hbm, idx_v, wbuf,
              sg0, sg1, so0, so1, si0, si1):
    cid = lax.axis_index("c")
    sid = lax.axis_index("s")
    wid = sid * NC + cid
    base = wid * TPW
    sg = (sg0, sg1)
    so = (so0, so1)
    si = (si0, si1)

    def tok0_of(c):
        return jnp.minimum(base + c * CH, N - CH)

    def issue_ids(c, b):
        pltpu.async_copy(ids_hbm.at[pl.ds(tok0_of(c), CH)], idx_v.at[b],
                         si[b])

    def wait_ids(b):
        pltpu.make_async_copy(ids_hbm.at[pl.ds(0, CH)], idx_v.at[b],
                              si[b]).wait()

    def issue_gather(b):
        pltpu.async_copy(word_hbm.at[idx_v.at[b]], wbuf.at[b], sg[b])

    def wait_gather(b):
        pltpu.make_async_copy(word_hbm.at[idx_v.at[b]], wbuf.at[b],
                              sg[b]).wait()

    def issue_out(c, b):
        pltpu.async_copy(wbuf.at[b], x_hbm.at[pl.ds(base + c * CH, CH)],
                         so[b])

    def wait_out(b):
        pltpu.make_async_copy(wbuf.at[b], x_hbm.at[pl.ds(0, CH)],
                              so[b]).wait()

    issue_ids(0, 0)
    issue_ids(1, 1)
    wait_ids(0)
    issue_gather(0)

    @pl.loop(0, NCH, step=2)
    def pair(c0):
        for bb in range(2):
            c = c0 + bb
            cur, nxt = bb, 1 - bb
            wait_gather(cur)
            wait_ids(nxt)
            if bb == 0:
                @pl.when(c0 >= 2)
                def _wait_prev_out():
                    wait_out(nxt)
            else:
                wait_out(nxt)
            issue_gather(nxt)
            issue_out(c, cur)
            issue_ids(c + 2, cur)

    wait_gather(0)
    wait_ids(1)
    wait_out(1)


def _tc_body(w_ref, p_ref, tt_ref, ty_ref, g_ref, b_ref, o_ref):
    tt = tt_ref[...]
    trow = jnp.where(tt == 0, ty_ref[0:1, :], ty_ref[1:2, :])
    x = w_ref[...] + p_ref[...] + trow
    mean = jnp.mean(x, axis=-1, keepdims=True)
    cx = x - mean
    var = jnp.mean(cx * cx, axis=-1, keepdims=True)
    o_ref[...] = cx * lax.rsqrt(var + EPS) * g_ref[...] + b_ref[...]


@jax.jit
def _bert_embed(ids_flat, tt_flat, word_table, pos_table, type_table,
                ln_gamma, ln_beta):
    mesh = plsc.VectorSubcoreMesh(core_axis_name="c", subcore_axis_name="s")
    kern = functools.partial(
        pl.kernel,
        mesh=mesh,
        compiler_params=pltpu.CompilerParams(needs_layout_passes=False),
        out_type=jax.ShapeDtypeStruct((N, HID), jnp.float32),
        scratch_types=[
            pltpu.VMEM((2, CH), jnp.int32),
            pltpu.VMEM((2, CH, HID), jnp.float32),
            pltpu.SemaphoreType.DMA,
            pltpu.SemaphoreType.DMA,
            pltpu.SemaphoreType.DMA,
            pltpu.SemaphoreType.DMA,
            pltpu.SemaphoreType.DMA,
            pltpu.SemaphoreType.DMA,
        ],
    )(_tec_body)
    w = kern(ids_flat, word_table)
    out = pl.pallas_call(
        _tc_body,
        grid=(S // BT, B),
        in_specs=[
            pl.BlockSpec((BT, HID), lambda j, b: (b * (S // BT) + j, 0)),
            pl.BlockSpec((BT, HID), lambda j, b: (j, 0)),
            pl.BlockSpec((BT, 1), lambda j, b: (b * (S // BT) + j, 0)),
            pl.BlockSpec((2, HID), lambda j, b: (0, 0)),
            pl.BlockSpec((1, HID), lambda j, b: (0, 0)),
            pl.BlockSpec((1, HID), lambda j, b: (0, 0)),
        ],
        out_specs=pl.BlockSpec((BT, HID), lambda j, b: (b * (S // BT) + j, 0)),
        out_shape=jax.ShapeDtypeStruct((N, HID), jnp.float32),
    )(w, pos_table, tt_flat.reshape(N, 1), type_table,
      ln_gamma.reshape(1, HID), ln_beta.reshape(1, HID))
    return out


def kernel(input_ids, token_type_ids, word_table, pos_table, type_table,
           ln_gamma, ln_beta):
    ids_flat = input_ids.reshape(-1).astype(jnp.int32)
    tt_flat = token_type_ids.reshape(-1).astype(jnp.int32)
    out = _bert_embed(ids_flat, tt_flat, word_table, pos_table,
                      type_table, ln_gamma, ln_beta)
    return out.reshape(B, S, HID)


# speedup vs baseline: 3.9503x; 1.0011x over previous
import functools

import jax
import jax.numpy as jnp
from jax import lax
from jax.experimental import pallas as pl
from jax.experimental.pallas import tpu as pltpu
from jax.experimental.pallas import tpu_sc as plsc

VOCAB = 100000
HID = 768
B = 4
S = 8192
EPS = 1e-12

N = B * S
NC = 2
NS = 16
NW = NC * NS
TPW = N // NW
CH = 32
NCH = TPW // CH
NBUF = 4
BT = 2048


def _tec_body(ids_hbm, word_hbm, x_hbm, idx_v, wbuf,
              sg0, sg1, sg2, sg3, so0, so1, so2, so3,
              si0, si1, si2, si3):
    cid = lax.axis_index("c")
    sid = lax.axis_index("s")
    wid = sid * NC + cid
    base = wid * TPW
    sg = (sg0, sg1, sg2, sg3)
    so = (so0, so1, so2, so3)
    si = (si0, si1, si2, si3)

    def tok0_of(c):
        return jnp.minimum(base + c * CH, N - CH)

    def issue_ids(c, b):
        pltpu.async_copy(ids_hbm.at[pl.ds(tok0_of(c), CH)], idx_v.at[b],
                         si[b])

    def wait_ids(b):
        pltpu.make_async_copy(ids_hbm.at[pl.ds(0, CH)], idx_v.at[b],
                              si[b]).wait()

    def issue_gather(b):
        pltpu.async_copy(word_hbm.at[idx_v.at[b]], wbuf.at[b], sg[b])

    def wait_gather(b):
        pltpu.make_async_copy(word_hbm.at[idx_v.at[b]], wbuf.at[b],
                              sg[b]).wait()

    def issue_out(c, b):
        pltpu.async_copy(wbuf.at[b], x_hbm.at[pl.ds(base + c * CH, CH)],
                         so[b])

    def wait_out(b):
        pltpu.make_async_copy(wbuf.at[b], x_hbm.at[pl.ds(0, CH)],
                              so[b]).wait()

    for k in range(NBUF):
        issue_ids(k, k)
    wait_ids(0)
    issue_gather(0)
    wait_ids(1)
    issue_gather(1)

    @pl.loop(0, NCH, step=NBUF)
    def quad(c0):
        for bb in range(NBUF):
            c = c0 + bb
            cur = bb
            pre = (bb + 2) % NBUF
            wait_gather(cur)
            wait_ids(pre)
            if bb < 2:
                @pl.when(c0 >= NBUF)
                def _wait_prev_out():
                    wait_out(pre)
            else:
                wait_out(pre)
            issue_gather(pre)
            issue_out(c, cur)
            issue_ids(c + NBUF, cur)
    wait_gather(0)
    wait_gather(1)
    wait_ids(2)
    wait_ids(3)
    wait_out(2)
    wait_out(3)


def _tc_body(w_ref, p_ref, tt_ref, ty_ref, g_ref, b_ref, o_ref):
    tt = tt_ref[...]
    trow = jnp.where(tt == 0, ty_ref[0:1, :], ty_ref[1:2, :])
    x = w_ref[...] + p_ref[...] + trow
    mean = jnp.mean(x, axis=-1, keepdims=True)
    cx = x - mean
    var = jnp.mean(cx * cx, axis=-1, keepdims=True)
    o_ref[...] = cx * lax.rsqrt(var + EPS) * g_ref[...] + b_ref[...]


@jax.jit
def _bert_embed(ids_flat, tt_flat, word_table, pos_table, type_table,
                ln_gamma, ln_beta):
    mesh = plsc.VectorSubcoreMesh(core_axis_name="c", subcore_axis_name="s")
    kern = functools.partial(
        pl.kernel,
        mesh=mesh,
        compiler_params=pltpu.CompilerParams(needs_layout_passes=False),
        out_type=jax.ShapeDtypeStruct((N, HID), jnp.float32),
        scratch_types=(
            [pltpu.VMEM((NBUF, CH), jnp.int32),
             pltpu.VMEM((NBUF, CH, HID), jnp.float32)]
            + [pltpu.SemaphoreType.DMA] * (3 * NBUF)),
    )(_tec_body)
    w = kern(ids_flat, word_table)
    out = pl.pallas_call(
        _tc_body,
        grid=(S // BT, B),
        in_specs=[
            pl.BlockSpec((BT, HID), lambda j, b: (b * (S // BT) + j, 0)),
            pl.BlockSpec((BT, HID), lambda j, b: (j, 0)),
            pl.BlockSpec((BT, 1), lambda j, b: (b * (S // BT) + j, 0)),
            pl.BlockSpec((2, HID), lambda j, b: (0, 0)),
            pl.BlockSpec((1, HID), lambda j, b: (0, 0)),
            pl.BlockSpec((1, HID), lambda j, b: (0, 0)),
        ],
        out_specs=pl.BlockSpec((BT, HID), lambda j, b: (b * (S // BT) + j, 0)),
        out_shape=jax.ShapeDtypeStruct((N, HID), jnp.float32),
    )(w, pos_table, tt_flat.reshape(N, 1), type_table,
      ln_gamma.reshape(1, HID), ln_beta.reshape(1, HID))
    return out


def kernel(input_ids, token_type_ids, word_table, pos_table, type_table,
           ln_gamma, ln_beta):
    ids_flat = input_ids.reshape(-1).astype(jnp.int32)
    tt_flat = token_type_ids.reshape(-1).astype(jnp.int32)
    out = _bert_embed(ids_flat, tt_flat, word_table, pos_table,
                      type_table, ln_gamma, ln_beta)
    return out.reshape(B, S, HID)
